# baseline (speedup 1.0000x reference)
import jax
import jax.numpy as jnp
from jax.experimental import pallas as pl
from jax.experimental.pallas import tpu as pltpu

B, N, CHIN, K, CM, L, HID = 8, 64, 12, 256, 16, 6, 32
BN = B * N


def _sig(x):
    return 0.5 * jnp.tanh(0.5 * x) + 0.5


def _swish(x):
    return x * _sig(x)


def _dot(a, b):
    return jax.lax.dot_general(
        a, b, (((1,), (0,)), ((), ())), preferred_element_type=jnp.float32
    )


def _layer_kernel(coords_ref, feats_ref, win_ref, bin_ref, w1_ref, b1_ref,
                  w2_ref, b2_ref, wl_ref, bl_ref, gamma_ref, beta_ref,
                  wot_ref, bout_ref, out_ref, h_s, pen_s, kwt_s):
    l = pl.program_id(0)
    cur = jax.lax.rem(l, 2)
    nxt = 1 - cur

    def weightnet(layer_idx, slot):
        w1b = w1_ref[layer_idx, 3:6, :]
        b1 = b1_ref[layer_idx].reshape(1, 1, HID)
        w2 = w2_ref[layer_idx]
        b2 = b2_ref[layer_idx]
        coords = coords_ref[...].reshape(BN, 3)
        p = _dot(coords, w1b)
        for b in range(B):
            pb = p[b * N:(b + 1) * N, :]
            kwp = pb[None, :, :] - pb[:, None, :] + b1
            a = _swish(kwp).reshape(N * N, HID)
            kw = (_dot(a, w2) + b2) * (1.0 / N)
            kwt = jnp.transpose(kw.reshape(N, N, CM), (0, 2, 1))
            kwt_s[slot, b] = kwt.reshape(N * CM, N)

    @pl.when(l == 0)
    def _init():
        feats = feats_ref[...].reshape(BN, CHIN)
        h_s[...] = _dot(feats, win_ref[...]) + bin_ref[...]
        weightnet(0, cur)

    for b in range(B):
        pen = _dot(kwt_s[cur, b], h_s[pl.ds(b * N, N), :])
        pen_s[pl.ds(b * N, N), :, :] = pen.reshape(N, CM, K)

    weightnet(jnp.minimum(l + 1, L - 1), nxt)

    acc = jnp.broadcast_to(bl_ref[0], (BN, K))
    for m in range(CM):
        acc = acc + _dot(pen_s[:, m, :], wl_ref[0, pl.ds(m * K, K), :])
    h_s[...] = acc

    @pl.when(l == L - 1)
    def _tail():
        h = acc
        mu = jnp.mean(h, axis=0, keepdims=True)
        var = jnp.mean((h - mu) ** 2, axis=0, keepdims=True)
        hn = (h - mu) * jax.lax.rsqrt(var + 1e-5) * gamma_ref[...] + beta_ref[...]
        hn = _swish(hn)
        s = jnp.sum(hn * wot_ref[...], axis=1, keepdims=True)
        o = _sig(s + bout_ref[0, 0])
        pooled = jnp.sum(o.reshape(B, N, 1), axis=1) * (1.0 / N)
        out_ref[...] = jnp.broadcast_to(pooled, (B, 128))


def kernel(coords, feats, mask, W_in, b_in, W1s, b1s, W2s, b2s, Wls, bls,
           gamma, beta, W_out, b_out):
    del mask
    b1s3 = b1s.reshape(L, 1, HID)
    b2s3 = b2s.reshape(L, 1, CM)
    bls3 = bls.reshape(L, 1, K)
    b_in2 = b_in.reshape(1, K)
    gamma2 = gamma.reshape(1, K)
    beta2 = beta.reshape(1, K)
    wot = W_out.reshape(1, K)
    bout = b_out.reshape(1, 1)

    grid = (L,)
    out = pl.pallas_call(
        _layer_kernel,
        grid=grid,
        in_specs=[
            pl.BlockSpec((B, N, 3), lambda l: (0, 0, 0)),
            pl.BlockSpec((B, N, CHIN), lambda l: (0, 0, 0)),
            pl.BlockSpec((CHIN, K), lambda l: (0, 0)),
            pl.BlockSpec((1, K), lambda l: (0, 0)),
            pl.BlockSpec((L, 6, HID), lambda l: (0, 0, 0)),
            pl.BlockSpec((L, 1, HID), lambda l: (0, 0, 0)),
            pl.BlockSpec((L, HID, CM), lambda l: (0, 0, 0)),
            pl.BlockSpec((L, 1, CM), lambda l: (0, 0, 0)),
            pl.BlockSpec((1, CM * K, K), lambda l: (l, 0, 0)),
            pl.BlockSpec((1, 1, K), lambda l: (l, 0, 0)),
            pl.BlockSpec((1, K), lambda l: (0, 0)),
            pl.BlockSpec((1, K), lambda l: (0, 0)),
            pl.BlockSpec((1, K), lambda l: (0, 0)),
            pl.BlockSpec((1, 1), lambda l: (0, 0)),
        ],
        out_specs=pl.BlockSpec((B, 128), lambda l: (0, 0)),
        out_shape=jax.ShapeDtypeStruct((B, 128), jnp.float32),
        scratch_shapes=[
            pltpu.VMEM((BN, K), jnp.float32),
            pltpu.VMEM((BN, CM, K), jnp.float32),
            pltpu.VMEM((2, B, N * CM, N), jnp.float32),
        ],
    )(coords, feats, W_in, b_in2, W1s, b1s3, W2s, b2s3, Wls, bls3,
      gamma2, beta2, wot, bout)
    return out[:, :1]


# speedup vs baseline: 1.0650x; 1.0650x over previous
import jax
import jax.numpy as jnp
from jax.experimental import pallas as pl
from jax.experimental.pallas import tpu as pltpu

B, N, CHIN, K, CM, L, HID = 8, 64, 12, 256, 16, 6, 32
BN = B * N


def _sig(x):
    return 0.5 * jnp.tanh(0.5 * x) + 0.5


def _swish(x):
    return x * _sig(x)


def _dot(a, b):
    return jax.lax.dot_general(
        a, b, (((1,), (0,)), ((), ())), preferred_element_type=jnp.float32
    )


def _dot_t(a, b):
    return jax.lax.dot_general(
        a, b, (((0,), (0,)), ((), ())), preferred_element_type=jnp.float32
    )


def _layer_kernel(coords_ref, feats_ref, win_ref, bin_ref, w1_ref, b1_ref,
                  w2_ref, b2_ref, wl_ref, bl_ref, gamma_ref, beta_ref,
                  wot_ref, bout_ref, out_ref, h_s, pen_s, kw_s):
    l = pl.program_id(0)

    @pl.when(l == 0)
    def _init():
        feats = feats_ref[...].reshape(BN, CHIN)
        h_s[...] = _dot(feats, win_ref[...]) + bin_ref[...]

    w1b = w1_ref[0, 3:6, :]
    b1 = b1_ref[0].reshape(1, 1, HID)
    w2 = w2_ref[0]
    b2 = b2_ref[0]
    coords = coords_ref[...].reshape(BN, 3)
    p = _dot(coords, w1b)
    for b in range(B):
        pb = p[b * N:(b + 1) * N, :]
        kwp = pb[:, None, :] - pb[None, :, :] + b1
        a = _swish(kwp).reshape(N * N, HID)
        kw = (_dot(a, w2) + b2) * (1.0 / N)
        kwt = jnp.transpose(kw.reshape(N, N, CM), (0, 2, 1))
        for m in range(CM):
            kw_s[b, m] = kwt[:, m, :]

    for b in range(B):
        hb = h_s[pl.ds(b * N, N), :]
        for m in range(CM):
            pen_s[pl.ds(b * N, N), pl.ds(m * K, K)] = _dot_t(kw_s[b, m], hb)

    acc = jnp.broadcast_to(bl_ref[0], (BN, K)) + _dot(pen_s[...], wl_ref[0])
    h_s[...] = acc

    @pl.when(l == L - 1)
    def _tail():
        h = acc
        mu = jnp.mean(h, axis=0, keepdims=True)
        var = jnp.mean((h - mu) ** 2, axis=0, keepdims=True)
        hn = (h - mu) * jax.lax.rsqrt(var + 1e-5) * gamma_ref[...] + beta_ref[...]
        hn = _swish(hn)
        s = jnp.sum(hn * wot_ref[...], axis=1, keepdims=True)
        o = _sig(s + bout_ref[0, 0])
        pooled = jnp.sum(o.reshape(B, N, 1), axis=1) * (1.0 / N)
        out_ref[...] = jnp.broadcast_to(pooled, (B, 128))


def kernel(coords, feats, mask, W_in, b_in, W1s, b1s, W2s, b2s, Wls, bls,
           gamma, beta, W_out, b_out):
    del mask
    b1s3 = b1s.reshape(L, 1, HID)
    b2s3 = b2s.reshape(L, 1, CM)
    bls3 = bls.reshape(L, 1, K)
    b_in2 = b_in.reshape(1, K)
    gamma2 = gamma.reshape(1, K)
    beta2 = beta.reshape(1, K)
    wot = W_out.reshape(1, K)
    bout = b_out.reshape(1, 1)

    grid = (L,)
    out = pl.pallas_call(
        _layer_kernel,
        grid=grid,
        in_specs=[
            pl.BlockSpec((B, N, 3), lambda l: (0, 0, 0)),
            pl.BlockSpec((B, N, CHIN), lambda l: (0, 0, 0)),
            pl.BlockSpec((CHIN, K), lambda l: (0, 0)),
            pl.BlockSpec((1, K), lambda l: (0, 0)),
            pl.BlockSpec((1, 6, HID), lambda l: (l, 0, 0)),
            pl.BlockSpec((1, 1, HID), lambda l: (l, 0, 0)),
            pl.BlockSpec((1, HID, CM), lambda l: (l, 0, 0)),
            pl.BlockSpec((1, 1, CM), lambda l: (l, 0, 0)),
            pl.BlockSpec((1, CM * K, K), lambda l: (l, 0, 0)),
            pl.BlockSpec((1, 1, K), lambda l: (l, 0, 0)),
            pl.BlockSpec((1, K), lambda l: (0, 0)),
            pl.BlockSpec((1, K), lambda l: (0, 0)),
            pl.BlockSpec((1, K), lambda l: (0, 0)),
            pl.BlockSpec((1, 1), lambda l: (0, 0)),
        ],
        out_specs=pl.BlockSpec((B, 128), lambda l: (0, 0)),
        out_shape=jax.ShapeDtypeStruct((B, 128), jnp.float32),
        scratch_shapes=[
            pltpu.VMEM((BN, K), jnp.float32),
            pltpu.VMEM((BN, CM * K), jnp.float32),
            pltpu.VMEM((B, CM, N, N), jnp.float32),
        ],
    )(coords, feats, W_in, b_in2, W1s, b1s3, W2s, b2s3, Wls, bls3,
      gamma2, beta2, wot, bout)
    return out[:, :1]


# speedup vs baseline: 1.0966x; 1.0297x over previous
import jax
import jax.numpy as jnp
from jax.experimental import pallas as pl
from jax.experimental.pallas import tpu as pltpu

B, N, CHIN, K, CM, L, HID = 8, 64, 12, 256, 16, 6, 32
BN = B * N


def _sig(x):
    return 0.5 * jnp.tanh(0.5 * x) + 0.5


def _swish(x):
    return x * _sig(x)


def _swish_half(u):
    return u + u * jnp.tanh(u)


def _dot(a, b):
    return jax.lax.dot_general(
        a, b, (((1,), (0,)), ((), ())), preferred_element_type=jnp.float32
    )


def _dot_t(a, b):
    return jax.lax.dot_general(
        a, b, (((0,), (0,)), ((), ())), preferred_element_type=jnp.float32
    )


def _layer_kernel(coords_ref, feats_ref, win_ref, bin_ref, w1_ref, b1_ref,
                  w2_ref, b2_ref, wl_ref, bl_ref, gamma_ref, beta_ref,
                  wot_ref, bout_ref, out_ref, h_s, pen_s, kw_s):
    l = pl.program_id(0)

    @pl.when(l == 0)
    def _init():
        feats = feats_ref[...].reshape(BN, CHIN)
        h_s[...] = _dot(feats, win_ref[...]) + bin_ref[...]

    w1b = w1_ref[0, 3:6, :]
    b1 = b1_ref[0].reshape(1, 1, HID)
    w2 = w2_ref[0]
    b2 = b2_ref[0]
    coords = coords_ref[...].reshape(BN, 3)
    p = _dot(coords, w1b)
    for b in range(B):
        pb = p[b * N:(b + 1) * N, :]
        kwp = pb[:, None, :] - pb[None, :, :] + b1
        a = _swish_half(kwp).reshape(N * N, HID)
        kw = _dot(a, w2) + b2
        kw_s[b] = jnp.transpose(kw.reshape(N, N, CM), (0, 2, 1))

    for b in range(B):
        hb = h_s[pl.ds(b * N, N), :]
        for m in range(CM):
            pen_s[pl.ds(b * N, N), pl.ds(m * K, K)] = _dot_t(kw_s[b, :, m, :], hb)

    acc = jnp.broadcast_to(bl_ref[0], (BN, K)) + _dot(pen_s[...], wl_ref[0])
    h_s[...] = acc

    @pl.when(l == L - 1)
    def _tail():
        h = acc
        mu = jnp.mean(h, axis=0, keepdims=True)
        var = jnp.mean((h - mu) ** 2, axis=0, keepdims=True)
        hn = (h - mu) * jax.lax.rsqrt(var + 1e-5) * gamma_ref[...] + beta_ref[...]
        hn = _swish(hn)
        s = jnp.sum(hn * wot_ref[...], axis=1, keepdims=True)
        o = _sig(s + bout_ref[0, 0])
        pooled = jnp.sum(o.reshape(B, N, 1), axis=1) * (1.0 / N)
        out_ref[...] = jnp.broadcast_to(pooled, (B, 128))


def kernel(coords, feats, mask, W_in, b_in, W1s, b1s, W2s, b2s, Wls, bls,
           gamma, beta, W_out, b_out):
    del mask
    W1s = W1s * 0.5
    b1s = b1s * 0.5
    W2s = W2s * (1.0 / N)
    b2s = b2s * (1.0 / N)
    b1s3 = b1s.reshape(L, 1, HID)
    b2s3 = b2s.reshape(L, 1, CM)
    bls3 = bls.reshape(L, 1, K)
    b_in2 = b_in.reshape(1, K)
    gamma2 = gamma.reshape(1, K)
    beta2 = beta.reshape(1, K)
    wot = W_out.reshape(1, K)
    bout = b_out.reshape(1, 1)

    grid = (L,)
    out = pl.pallas_call(
        _layer_kernel,
        grid=grid,
        in_specs=[
            pl.BlockSpec((B, N, 3), lambda l: (0, 0, 0)),
            pl.BlockSpec((B, N, CHIN), lambda l: (0, 0, 0)),
            pl.BlockSpec((CHIN, K), lambda l: (0, 0)),
            pl.BlockSpec((1, K), lambda l: (0, 0)),
            pl.BlockSpec((1, 6, HID), lambda l: (l, 0, 0)),
            pl.BlockSpec((1, 1, HID), lambda l: (l, 0, 0)),
            pl.BlockSpec((1, HID, CM), lambda l: (l, 0, 0)),
            pl.BlockSpec((1, 1, CM), lambda l: (l, 0, 0)),
            pl.BlockSpec((1, CM * K, K), lambda l: (l, 0, 0)),
            pl.BlockSpec((1, 1, K), lambda l: (l, 0, 0)),
            pl.BlockSpec((1, K), lambda l: (0, 0)),
            pl.BlockSpec((1, K), lambda l: (0, 0)),
            pl.BlockSpec((1, K), lambda l: (0, 0)),
            pl.BlockSpec((1, 1), lambda l: (0, 0)),
        ],
        out_specs=pl.BlockSpec((B, 128), lambda l: (0, 0)),
        out_shape=jax.ShapeDtypeStruct((B, 128), jnp.float32),
        scratch_shapes=[
            pltpu.VMEM((BN, K), jnp.float32),
            pltpu.VMEM((BN, CM * K), jnp.float32),
            pltpu.VMEM((B, N, CM, N), jnp.float32),
        ],
    )(coords, feats, W_in, b_in2, W1s, b1s3, W2s, b2s3, Wls, bls3,
      gamma2, beta2, wot, bout)
    return out[:, :1]
